# baseline (speedup 1.0000x reference)
import functools

import jax
import jax.numpy as jnp
from jax import lax
from jax.experimental import pallas as pl
from jax.experimental.pallas import tpu as pltpu
from jax.experimental.pallas import tpu_sc as plsc

N_NODES = 100000
N_EDGES = 1600000
D_IN = 32
D_OUT = 2
NUM_REL = 2

G = 8
NC = 2
NS = 16
NW = NC * NS
EDGES_PER_TILE = N_EDGES // NW
CHUNK = 2000
ROW_BLK = 2048
NPAD = 102400
ACC_ROWS = NUM_REL * NPAD
PK = ROW_BLK * G // 128
NB = NPAD // ROW_BLK


def _project_body(x_ref, w_ref, o_ref, z_ref):
    y = jnp.dot(x_ref[...], w_ref[...], preferred_element_type=jnp.float32)
    b = y.shape[0]
    ones = jnp.ones((b, 1), jnp.float32)
    zeros = jnp.zeros((b, G - D_OUT - 1), jnp.float32)
    zer3 = jnp.zeros((b, 3), jnp.float32)
    zend = jnp.zeros((b, G - 3 - D_OUT), jnp.float32)
    r0 = jnp.concatenate([y[:, 0:D_OUT], ones, zeros], axis=1)
    r1 = jnp.concatenate([y[:, D_OUT:2 * D_OUT], ones, zeros], axis=1)
    z0 = jnp.concatenate(
        [zer3, 0.5 * y[:, 2 * D_OUT:3 * D_OUT], zend], axis=1)
    rr0 = r0.reshape(b // 16, 16, G)
    rr1 = r1.reshape(b // 16, 16, G)
    zz0 = z0.reshape(b // 16, 16, G)
    for k in range(16):
        o_ref[0, :, k * G:(k + 1) * G] = rr0[:, k, :]
        o_ref[1, :, k * G:(k + 1) * G] = rr1[:, k, :]
        z_ref[0, :, k * G:(k + 1) * G] = zz0[:, k, :]
    z_ref[1] = jnp.zeros((b * G // 128, 128), jnp.float32)


def _project(x, w_full):
    shp = jax.ShapeDtypeStruct((NUM_REL, NPAD * G // 128, 128), jnp.float32)
    zshp = jax.ShapeDtypeStruct((NUM_REL, NPAD * G // 128, 128), jnp.float32)
    return pl.pallas_call(
        _project_body,
        grid=(NB,),
        in_specs=[
            pl.BlockSpec((ROW_BLK, D_IN), lambda i: (i, 0)),
            pl.BlockSpec((D_IN, (NUM_REL + 1) * D_OUT), lambda i: (0, 0)),
        ],
        out_specs=(pl.BlockSpec((NUM_REL, PK, 128), lambda i: (0, i, 0)),
                   pl.BlockSpec((NUM_REL, PK, 128), lambda i: (0, i, 0))),
        out_shape=(shp, zshp),
    )(x, w_full)


def _sc_body(y_hbm, ei_hbm, et_hbm, z_hbm, o_hbm,
             sbuf, dbuf, tbuf, gv, sv, rows0, rows1,
             acc, gsem0, gsem1, ssem0, ssem1):
    cid = lax.axis_index("c")
    sid = lax.axis_index("s")
    wid = sid * NC + cid
    rows = (rows0, rows1)
    gsem = (gsem0, gsem1)
    ssem = (ssem0, ssem1)

    rows_per_sub = ACC_ROWS // NS
    zbase = sid * rows_per_sub
    pltpu.sync_copy(z_hbm.at[pl.ds(zbase, rows_per_sub)],
                    acc.at[pl.ds(zbase, rows_per_sub)])
    plsc.subcore_barrier()

    ebase = wid * EDGES_PER_TILE
    H = CHUNK // 2

    @pl.loop(0, EDGES_PER_TILE, step=CHUNK)
    def _(c):
        b = ebase + c
        pltpu.sync_copy(ei_hbm.at[0, pl.ds(b, CHUNK)], sbuf)
        pltpu.sync_copy(ei_hbm.at[1, pl.ds(b, CHUNK)], dbuf)
        pltpu.sync_copy(et_hbm.at[pl.ds(b, CHUNK)], tbuf)

        @pl.loop(0, CHUNK, step=16)
        def _(i):
            t = tbuf[pl.ds(i, 16)] * NPAD
            gv[pl.ds(i, 16)] = t + sbuf[pl.ds(i, 16)]
            sv[pl.ds(i, 16)] = t + dbuf[pl.ds(i, 16)]

        g0 = pltpu.async_copy(y_hbm.at[gv.at[pl.ds(0, H)]], rows[0], gsem[0])
        g0.wait()
        s0 = pltpu.async_copy(rows[0], acc.at[sv.at[pl.ds(0, H)]],
                              ssem[0], add=True)
        g1 = pltpu.async_copy(y_hbm.at[gv.at[pl.ds(H, H)]], rows[1], gsem[1])
        g1.wait()
        s0.wait()
        s1 = pltpu.async_copy(rows[1], acc.at[sv.at[pl.ds(H, H)]],
                              ssem[1], add=True)
        s1.wait()

    plsc.subcore_barrier()
    pltpu.sync_copy(acc.at[pl.ds(zbase, rows_per_sub)],
                    o_hbm.at[cid].at[pl.ds(zbase, rows_per_sub)])


def _sc_aggregate(y_flat, ei32, et32, zinit):
    mesh = plsc.VectorSubcoreMesh(core_axis_name="c", subcore_axis_name="s")
    kern = functools.partial(
        pl.kernel,
        mesh=mesh,
        compiler_params=pltpu.CompilerParams(use_tc_tiling_on_sc=False),
        out_type=jax.ShapeDtypeStruct((NC, ACC_ROWS, G), jnp.float32),
        scratch_types=(
            [pltpu.VMEM((CHUNK,), jnp.int32)] * 3
            + [pltpu.VMEM((CHUNK,), jnp.int32)] * 2
            + [pltpu.VMEM((CHUNK // 2, G), jnp.float32)] * 2
            + [pltpu.VMEM_SHARED((ACC_ROWS, G), jnp.float32)]
            + [pltpu.SemaphoreType.DMA] * 4
        ),
    )(_sc_body)
    return kern(y_flat, ei32, et32, zinit)


def _final_body(a0_ref, a1_ref, o_ref):
    i128 = lax.broadcasted_iota(jnp.int32, (128, 128), 0)
    j128 = lax.broadcasted_iota(jnp.int32, (128, 128), 1)
    csel = ((j128 // G) * G + 2 == i128).astype(jnp.float32)
    dmask = (j128[0:1, :] % G) < D_OUT
    i16 = lax.broadcasted_iota(jnp.int32, (128, 16), 0)
    j16 = lax.broadcasted_iota(jnp.int32, (128, 16), 1)
    esel = [(i16 == G * j16 + c).astype(jnp.float32) for c in range(D_OUT)]
    rsel = [(i16 == G * j16 + 3 + c).astype(jnp.float32) for c in range(D_OUT)]

    ch = [None] * D_OUT
    for rel, a_ref in enumerate((a0_ref, a1_ref)):
        a = a_ref[...]
        s = a[0] + a[1]
        cntb = jnp.dot(s, csel, preferred_element_type=jnp.float32)
        m = jnp.where(dmask, s / jnp.maximum(cntb, 1.0), s)
        for c in range(D_OUT):
            sel = esel[c] + rsel[c] if rel == 0 else esel[c]
            v = jnp.dot(m, sel, preferred_element_type=jnp.float32)
            ch[c] = v if ch[c] is None else ch[c] + v
    for c in range(D_OUT):
        o_ref[c] = ch[c]


def _final(acc128):
    return pl.pallas_call(
        _final_body,
        grid=(NB,),
        in_specs=[
            pl.BlockSpec((NC, PK, 128), lambda i: (0, i, 0)),
            pl.BlockSpec((NC, PK, 128), lambda i: (0, NB + i, 0)),
        ],
        out_specs=pl.BlockSpec((D_OUT, PK, 16), lambda i: (0, i, 0)),
        out_shape=jax.ShapeDtypeStruct(
            (D_OUT, NPAD * G // 128, 16), jnp.float32),
    )(acc128, acc128)


def kernel(x, edge_index, edge_type, weight, root):
    ei32 = edge_index.astype(jnp.int32)
    et32 = edge_type.astype(jnp.int32)
    w_full = jnp.concatenate([weight[0], weight[1], root], axis=1)

    x_pad = jnp.pad(x, ((0, NPAD - N_NODES), (0, 0)))
    ypk, zpk = _project(x_pad, w_full)
    y_flat = ypk.reshape(NUM_REL * NPAD, G)
    zinit = zpk.reshape(ACC_ROWS, G)
    acc = _sc_aggregate(y_flat, ei32, et32, zinit)
    acc128 = acc.reshape(NC, ACC_ROWS * G // 128, 128)
    out_t = _final(acc128)
    return out_t.reshape(D_OUT, NPAD)[:, :N_NODES].T


# speedup vs baseline: 1.0252x; 1.0252x over previous
import functools

import jax
import jax.numpy as jnp
from jax import lax
from jax.experimental import pallas as pl
from jax.experimental.pallas import tpu as pltpu
from jax.experimental.pallas import tpu_sc as plsc

N_NODES = 100000
N_EDGES = 1600000
D_IN = 32
D_OUT = 2
NUM_REL = 2

G = 8
NC = 2
NS = 16
NW = NC * NS
EDGES_PER_TILE = N_EDGES // NW
CHUNK = 2000
ROW_BLK = 2048
NPAD = 102400
ACC_ROWS = NUM_REL * NPAD
PK = ROW_BLK * G // 128
NB = NPAD // ROW_BLK


def _project_body(x_ref, w_ref, o_ref, z_ref):
    y = jnp.dot(x_ref[...], w_ref[...], preferred_element_type=jnp.float32)
    b = y.shape[0]
    ones = jnp.ones((b, 1), jnp.float32)
    zeros = jnp.zeros((b, G - D_OUT - 1), jnp.float32)
    zer3 = jnp.zeros((b, 3), jnp.float32)
    zend = jnp.zeros((b, G - 3 - D_OUT), jnp.float32)
    r0 = jnp.concatenate([y[:, 0:D_OUT], ones, zeros], axis=1)
    r1 = jnp.concatenate([y[:, D_OUT:2 * D_OUT], ones, zeros], axis=1)
    z0 = jnp.concatenate(
        [zer3, 0.5 * y[:, 2 * D_OUT:3 * D_OUT], zend], axis=1)
    rr0 = r0.reshape(b // 16, 16, G)
    rr1 = r1.reshape(b // 16, 16, G)
    zz0 = z0.reshape(b // 16, 16, G)
    for k in range(16):
        o_ref[0, :, k * G:(k + 1) * G] = rr0[:, k, :]
        o_ref[1, :, k * G:(k + 1) * G] = rr1[:, k, :]
        z_ref[0, :, k * G:(k + 1) * G] = zz0[:, k, :]
    z_ref[1] = jnp.zeros((b * G // 128, 128), jnp.float32)


def _project(x, w_full):
    shp = jax.ShapeDtypeStruct((NUM_REL, NPAD * G // 128, 128), jnp.float32)
    zshp = jax.ShapeDtypeStruct((NUM_REL, NPAD * G // 128, 128), jnp.float32)
    return pl.pallas_call(
        _project_body,
        grid=(NB,),
        in_specs=[
            pl.BlockSpec((ROW_BLK, D_IN), lambda i: (i, 0)),
            pl.BlockSpec((D_IN, (NUM_REL + 1) * D_OUT), lambda i: (0, 0)),
        ],
        out_specs=(pl.BlockSpec((NUM_REL, PK, 128), lambda i: (0, i, 0)),
                   pl.BlockSpec((NUM_REL, PK, 128), lambda i: (0, i, 0))),
        out_shape=(shp, zshp),
    )(x, w_full)


def _sc_body(y_hbm, ei_hbm, et_hbm, z_hbm, o_hbm,
             sbuf, dbuf, tbuf, gv, sv, rows_v, acc, sem):
    cid = lax.axis_index("c")
    sid = lax.axis_index("s")
    wid = sid * NC + cid

    rows_per_sub = ACC_ROWS // NS
    zbase = sid * rows_per_sub
    pltpu.sync_copy(z_hbm.at[pl.ds(zbase, rows_per_sub)],
                    acc.at[pl.ds(zbase, rows_per_sub)])
    plsc.subcore_barrier()

    ebase = wid * EDGES_PER_TILE

    @pl.loop(0, EDGES_PER_TILE, step=CHUNK)
    def _(c):
        b = ebase + c
        pltpu.sync_copy(ei_hbm.at[0, pl.ds(b, CHUNK)], sbuf)
        pltpu.sync_copy(ei_hbm.at[1, pl.ds(b, CHUNK)], dbuf)
        pltpu.sync_copy(et_hbm.at[pl.ds(b, CHUNK)], tbuf)

        @pl.loop(0, CHUNK, step=16)
        def _(i):
            t = tbuf[pl.ds(i, 16)] * NPAD
            gv[pl.ds(i, 16)] = t + sbuf[pl.ds(i, 16)]
            sv[pl.ds(i, 16)] = t + dbuf[pl.ds(i, 16)]

        pltpu.async_copy(y_hbm.at[gv], rows_v, sem).wait()
        pltpu.sync_copy(rows_v, acc.at[sv], add=True)

    plsc.subcore_barrier()
    pltpu.sync_copy(acc.at[pl.ds(zbase, rows_per_sub)],
                    o_hbm.at[cid].at[pl.ds(zbase, rows_per_sub)])


def _sc_aggregate(y_flat, ei32, et32, zinit):
    mesh = plsc.VectorSubcoreMesh(core_axis_name="c", subcore_axis_name="s")
    kern = functools.partial(
        pl.kernel,
        mesh=mesh,
        compiler_params=pltpu.CompilerParams(use_tc_tiling_on_sc=False),
        out_type=jax.ShapeDtypeStruct((NC, ACC_ROWS, G), jnp.float32),
        scratch_types=[
            pltpu.VMEM((CHUNK,), jnp.int32),
            pltpu.VMEM((CHUNK,), jnp.int32),
            pltpu.VMEM((CHUNK,), jnp.int32),
            pltpu.VMEM((CHUNK,), jnp.int32),
            pltpu.VMEM((CHUNK,), jnp.int32),
            pltpu.VMEM((CHUNK, G), jnp.float32),
            pltpu.VMEM_SHARED((ACC_ROWS, G), jnp.float32),
            pltpu.SemaphoreType.DMA,
        ],
    )(_sc_body)
    return kern(y_flat, ei32, et32, zinit)


def _final_body(a0_ref, a1_ref, o_ref):
    i128 = lax.broadcasted_iota(jnp.int32, (128, 128), 0)
    j128 = lax.broadcasted_iota(jnp.int32, (128, 128), 1)
    csel = ((j128 // G) * G + 2 == i128).astype(jnp.float32)
    dmask = (j128[0:1, :] % G) < D_OUT
    i16 = lax.broadcasted_iota(jnp.int32, (128, 16), 0)
    j16 = lax.broadcasted_iota(jnp.int32, (128, 16), 1)
    esel = [(i16 == G * j16 + c).astype(jnp.float32) for c in range(D_OUT)]
    rsel = [(i16 == G * j16 + 3 + c).astype(jnp.float32) for c in range(D_OUT)]

    ch = [None] * D_OUT
    for rel, a_ref in enumerate((a0_ref, a1_ref)):
        a = a_ref[...]
        s = a[0] + a[1]
        cntb = jnp.dot(s, csel, preferred_element_type=jnp.float32)
        m = jnp.where(dmask, s / jnp.maximum(cntb, 1.0), s)
        for c in range(D_OUT):
            sel = esel[c] + rsel[c] if rel == 0 else esel[c]
            v = jnp.dot(m, sel, preferred_element_type=jnp.float32)
            ch[c] = v if ch[c] is None else ch[c] + v
    for c in range(D_OUT):
        o_ref[c] = ch[c]


def _final(acc128):
    return pl.pallas_call(
        _final_body,
        grid=(NB,),
        in_specs=[
            pl.BlockSpec((NC, PK, 128), lambda i: (0, i, 0)),
            pl.BlockSpec((NC, PK, 128), lambda i: (0, NB + i, 0)),
        ],
        out_specs=pl.BlockSpec((D_OUT, PK, 16), lambda i: (0, i, 0)),
        out_shape=jax.ShapeDtypeStruct(
            (D_OUT, NPAD * G // 128, 16), jnp.float32),
    )(acc128, acc128)


def kernel(x, edge_index, edge_type, weight, root):
    ei32 = edge_index.astype(jnp.int32)
    et32 = edge_type.astype(jnp.int32)
    w_full = jnp.concatenate([weight[0], weight[1], root], axis=1)

    x_pad = jnp.pad(x, ((0, NPAD - N_NODES), (0, 0)))
    ypk, zpk = _project(x_pad, w_full)
    y_flat = ypk.reshape(NUM_REL * NPAD, G)
    zinit = zpk.reshape(ACC_ROWS, G)
    acc = _sc_aggregate(y_flat, ei32, et32, zinit)
    acc128 = acc.reshape(NC, ACC_ROWS * G // 128, 128)
    out_t = _final(acc128)
    return out_t.reshape(D_OUT, NPAD)[:, :N_NODES].T
